# baseline (speedup 1.0000x reference)
import functools

import jax
import jax.numpy as jnp
from jax import lax
from jax.experimental import pallas as pl
from jax.experimental.pallas import tpu as pltpu
from jax.experimental.pallas import tpu_sc as plsc

M = 1000000
B = 16384
D = 64
L = 16
NC = 2
NS = 16
NW = NC * NS
R = 65536
MP = NS * R
QT = B // NW
GC = 128
DS = B // NS
DC = 2048
QH = QT // 4


def _body(mem_hbm, dst_hbm, msgs_hbm, q_hbm, out_hbm, win_hbm, dst_sh):
    c = lax.axis_index("c")
    s = lax.axis_index("s")
    wid = c * NS + s

    def phase_build(dst_v, win_v):
        sl = pl.ds(s * DS, DS)
        pltpu.sync_copy(dst_hbm.at[sl], dst_v.at[pl.ds(0, DS)])
        pltpu.sync_copy(dst_v.at[pl.ds(0, DS)], dst_sh.at[sl])
        plsc.subcore_barrier()

        neg1 = jnp.full((L,), -1, dtype=jnp.int32)

        def memset_body(j, _):
            for k in range(8):
                win_v[pl.ds((j * 8 + k) * L, L)] = neg1
            return 0

        lax.fori_loop(0, R // (8 * L), memset_body, 0)

        lanes = lax.iota(jnp.int32, L)

        def scan_body(j, _):
            v = dst_v[pl.ds((j * L) % DC, L)]
            pos = lanes + j * L
            m = (v >> 16) == s
            local = jnp.where(m, v & 0xFFFF, 0)
            plsc.store_scatter(win_v, [local], pos, mask=m)
            g = plsc.load_gather(win_v, [local])
            need = m & (g < pos)

            def conflict_cond(carry):
                return jnp.any(carry)

            def conflict_body(carry):
                plsc.store_scatter(win_v, [local], pos, mask=carry)
                g2 = plsc.load_gather(win_v, [local])
                return m & (g2 < pos)

            lax.while_loop(conflict_cond, conflict_body, need)
            return 0

        for ci in range(B // DC):
            pltpu.sync_copy(dst_sh.at[pl.ds(ci * DC, DC)], dst_v)
            lax.fori_loop(ci * (DC // L), (ci + 1) * (DC // L), scan_body, 0)

        pltpu.sync_copy(win_v, win_hbm.at[pl.ds(c * MP + s * R, R)])

    with jax.named_scope("build_win"):
        pl.run_scoped(
            phase_build,
            pltpu.VMEM((DC,), jnp.int32),
            pltpu.VMEM((R,), jnp.int32),
        )

        plsc.subcore_barrier()

    def phase_query(q_v, wq_v, wq8_v, rows_a, rows_b, out_v, sem):
        base_q = wid * QT
        pltpu.sync_copy(q_hbm.at[pl.ds(base_q, QT)], q_v)

        with jax.named_scope("win_gather"):
            def wg_body(j, _):
                avec = ((q_v[pl.ds(j * L, L)] + c * MP) >> 3) << 3
                for k in range(L):
                    i = j * L + k
                    ak = pl.multiple_of(avec[k], 8)
                    pltpu.async_copy(
                        win_hbm.at[pl.ds(ak, 8)],
                        wq8_v.at[pl.ds(i * 8, 8)], sem)
                for k in range(L):
                    i = j * L + k
                    pltpu.make_async_copy(
                        win_hbm.at[pl.ds(0, 8)],
                        wq8_v.at[pl.ds(i * 8, 8)], sem).wait()
                return 0

            lax.fori_loop(0, QT // L, wg_body, 0)

            lanes0 = lax.iota(jnp.int32, L)

            def wx_body(j, _):
                sl = pl.ds(j * L, L)
                idx = (j * L + lanes0) * 8 + (q_v[sl] & 7)
                wq_v[sl] = plsc.load_gather(wq8_v, [idx])
                return 0

            lax.fori_loop(0, QT // L, wx_body, 0)

        lanes = lax.iota(jnp.int32, L)

        for h in range(QT // QH):
            with jax.named_scope("row_fetch"):
                def fetch_body(j, _):
                    sl = pl.ds(h * QH + j * L, L)
                    qvec = q_v[sl]
                    wvec = wq_v[sl]
                    mvec = jnp.where(
                        wvec >= 0, wvec, base_q + h * QH + j * L + lanes)
                    hi_vec = (qvec >= M // 2).astype(jnp.int32)
                    qh_vec = qvec - hi_vec * (M // 2)
                    for k in range(L):
                        i = j * L + k
                        pltpu.async_copy(
                            mem_hbm.at[pl.ds(qh_vec[k], 1)],
                            rows_a.at[pl.ds(i, 1)], sem)
                        pltpu.async_copy(
                            msgs_hbm.at[pl.ds(mvec[k], 1)],
                            rows_b.at[pl.ds(i, 1)], sem)
                    for k in range(L):
                        i = j * L + k
                        pltpu.make_async_copy(
                            mem_hbm.at[pl.ds(0, 1)],
                            rows_a.at[pl.ds(i, 1)], sem).wait()
                        pltpu.make_async_copy(
                            mem_hbm.at[pl.ds(0, 1)],
                            rows_b.at[pl.ds(i, 1)], sem).wait()
                    return 0

                lax.fori_loop(0, QH // L, fetch_body, 0)

            with jax.named_scope("select"):
                zrow = jnp.zeros((L,), dtype=jnp.float32)

                def select_body(j, _):
                    sl = pl.ds(h * QH + j * L, L)
                    wvec = wq_v[sl]
                    qvec = q_v[sl]
                    pqvec = (qvec >= M // 2).astype(jnp.int32) * D
                    for k in range(L):
                        valid = wvec[k] >= 0
                        pq = pqvec[k]
                        i = j * L + k
                        for dv in range(D // L):
                            a = rows_b[i, pl.ds(dv * L, L)]
                            b = rows_a[i, pl.ds(pq + dv * L, L)]
                            out_v[i, pl.ds(dv * L, L)] = jnp.where(
                                valid, a, b)
                        for dv in range(D // L):
                            out_v[i, pl.ds(D + dv * L, L)] = zrow
                    return 0

                lax.fori_loop(0, QH // L, select_body, 0)

            pltpu.sync_copy(out_v, out_hbm.at[pl.ds(base_q + h * QH, QH)])

    with jax.named_scope("serve_queries"):
        pl.run_scoped(
            phase_query,
            pltpu.VMEM((QT,), jnp.int32),
            pltpu.VMEM((QT,), jnp.int32),
            pltpu.VMEM((QT * 8,), jnp.int32),
            pltpu.VMEM((QH, 2 * D), jnp.float32),
            pltpu.VMEM((QH, D), jnp.float32),
            pltpu.VMEM((QH, 2 * D), jnp.float32),
            pltpu.SemaphoreType.DMA,
        )


def _fold_body(top_ref, bot_ref, o_ref):
    o_ref[:, :D] = top_ref[...]
    o_ref[:, D:] = bot_ref[...]


def _fold128(x):
    n = x.shape[0]
    blk = 4000
    nb = n // 2 // blk
    return pl.pallas_call(
        _fold_body,
        grid=(nb,),
        in_specs=[
            pl.BlockSpec((blk, D), lambda i: (i, 0)),
            pl.BlockSpec((blk, D), lambda i, _nb=nb: (i + _nb, 0)),
        ],
        out_specs=pl.BlockSpec((blk, 2 * D), lambda i: (i, 0)),
        out_shape=jax.ShapeDtypeStruct((n // 2, 2 * D), jnp.float32),
    )(x, x)


@jax.jit
def kernel(mem, dst_ids, msgs, query_ids):
    mesh = plsc.VectorSubcoreMesh(core_axis_name="c", subcore_axis_name="s")
    mem2 = _fold128(mem)
    out, _ = pl.kernel(
        _body,
        out_type=(
            jax.ShapeDtypeStruct((B, 2 * D), jnp.float32),
            jax.ShapeDtypeStruct((NC * MP,), jnp.int32),
        ),
        mesh=mesh,
        scratch_types=[pltpu.VMEM_SHARED((B,), jnp.int32)],
        compiler_params=pltpu.CompilerParams(
            needs_layout_passes=False, use_tc_tiling_on_sc=True),
    )(mem2, dst_ids, msgs, query_ids)
    return out[:, :D]


# speedup vs baseline: 1.5361x; 1.5361x over previous
import functools

import jax
import jax.numpy as jnp
from jax import lax
from jax.experimental import pallas as pl
from jax.experimental.pallas import tpu as pltpu
from jax.experimental.pallas import tpu_sc as plsc

M = 1000000
B = 16384
D = 64
L = 16
NC = 2
NS = 16
NW = NC * NS
R = 65536
MP = NS * R
QT = B // NW
GC = 128
DS = B // NS
DC = 2048
QH = QT // 4


def _body(mem_hbm, dst_hbm, msgs_hbm, q_hbm, out_hbm, win_hbm, dst_sh):
    c = lax.axis_index("c")
    s = lax.axis_index("s")
    wid = c * NS + s

    def phase_build(dst_v, win_v):
        sl = pl.ds(s * DS, DS)
        pltpu.sync_copy(dst_hbm.at[sl], dst_v.at[pl.ds(0, DS)])
        pltpu.sync_copy(dst_v.at[pl.ds(0, DS)], dst_sh.at[sl])
        plsc.subcore_barrier()

        neg1 = jnp.full((L,), -1, dtype=jnp.int32)

        def memset_body(j, _):
            for k in range(8):
                win_v[pl.ds((j * 8 + k) * L, L)] = neg1
            return 0

        lax.fori_loop(0, R // (8 * L), memset_body, 0)

        lanes = lax.iota(jnp.int32, L)

        def scan_body(j, _):
            v = dst_v[pl.ds((j * L) % DC, L)]
            pos = lanes + j * L
            m = (v >> 16) == s
            local = jnp.where(m, v & 0xFFFF, 0)
            plsc.store_scatter(win_v, [local], pos, mask=m)
            g = plsc.load_gather(win_v, [local])
            need = m & (g < pos)

            def conflict_cond(carry):
                return jnp.any(carry)

            def conflict_body(carry):
                plsc.store_scatter(win_v, [local], pos, mask=carry)
                g2 = plsc.load_gather(win_v, [local])
                return m & (g2 < pos)

            lax.while_loop(conflict_cond, conflict_body, need)
            return 0

        for ci in range(B // DC):
            pltpu.sync_copy(dst_sh.at[pl.ds(ci * DC, DC)], dst_v)
            lax.fori_loop(ci * (DC // L), (ci + 1) * (DC // L), scan_body, 0)

        pltpu.sync_copy(win_v, win_hbm.at[pl.ds(c * MP + s * R, R)])

    with jax.named_scope("build_win"):
        pl.run_scoped(
            phase_build,
            pltpu.VMEM((DC,), jnp.int32),
            pltpu.VMEM((R,), jnp.int32),
        )

        plsc.subcore_barrier()

    def phase_query(q_v, wq_v, wq8_v, rows_a, rows_b, out_v, sem):
        base_q = wid * QT
        pltpu.sync_copy(q_hbm.at[pl.ds(base_q, QT)], q_v)

        with jax.named_scope("win_gather"):
            def wg_body(j, _):
                avec = ((q_v[pl.ds(j * L, L)] + c * MP) >> 3) << 3
                for k in range(L):
                    i = j * L + k
                    ak = pl.multiple_of(avec[k], 8)
                    pltpu.async_copy(
                        win_hbm.at[pl.ds(ak, 8)],
                        wq8_v.at[pl.ds(i * 8, 8)], sem)
                for k in range(L):
                    i = j * L + k
                    pltpu.make_async_copy(
                        win_hbm.at[pl.ds(0, 8)],
                        wq8_v.at[pl.ds(i * 8, 8)], sem).wait()
                return 0

            lax.fori_loop(0, QT // L, wg_body, 0)

            lanes0 = lax.iota(jnp.int32, L)

            def wx_body(j, _):
                sl = pl.ds(j * L, L)
                idx = (j * L + lanes0) * 8 + (q_v[sl] & 7)
                wq_v[sl] = plsc.load_gather(wq8_v, [idx])
                return 0

            lax.fori_loop(0, QT // L, wx_body, 0)

        lanes = lax.iota(jnp.int32, L)

        for h in range(QT // QH):
            with jax.named_scope("row_fetch"):
                def fetch_body(j, _):
                    sl = pl.ds(h * QH + j * L, L)
                    qvec = q_v[sl]
                    wvec = wq_v[sl]
                    mvec = jnp.where(
                        wvec >= 0, wvec, base_q + h * QH + j * L + lanes)
                    for k in range(L):
                        i = j * L + k
                        pltpu.async_copy(
                            mem_hbm.at[pl.ds(qvec[k], 1)],
                            rows_a.at[pl.ds(i, 1)], sem)
                        pltpu.async_copy(
                            msgs_hbm.at[pl.ds(mvec[k], 1)],
                            rows_b.at[pl.ds(i, 1)], sem)
                    for k in range(L):
                        i = j * L + k
                        pltpu.make_async_copy(
                            mem_hbm.at[pl.ds(0, 1)],
                            rows_a.at[pl.ds(i, 1)], sem).wait()
                        pltpu.make_async_copy(
                            mem_hbm.at[pl.ds(0, 1)],
                            rows_b.at[pl.ds(i, 1)], sem).wait()
                    return 0

                lax.fori_loop(0, QH // L, fetch_body, 0)

            with jax.named_scope("select"):
                zrow = jnp.zeros((L,), dtype=jnp.float32)

                def select_body(j, _):
                    sl = pl.ds(h * QH + j * L, L)
                    wvec = wq_v[sl]
                    for k in range(L):
                        valid = wvec[k] >= 0
                        i = j * L + k
                        for dv in range(D // L):
                            a = rows_b[i, pl.ds(dv * L, L)]
                            b = rows_a[i, pl.ds(dv * L, L)]
                            out_v[i, pl.ds(dv * L, L)] = jnp.where(
                                valid, a, b)
                        for dv in range(D // L):
                            out_v[i, pl.ds(D + dv * L, L)] = zrow
                    return 0

                lax.fori_loop(0, QH // L, select_body, 0)

            pltpu.sync_copy(out_v, out_hbm.at[pl.ds(base_q + h * QH, QH)])

    with jax.named_scope("serve_queries"):
        pl.run_scoped(
            phase_query,
            pltpu.VMEM((QT,), jnp.int32),
            pltpu.VMEM((QT,), jnp.int32),
            pltpu.VMEM((QT * 8,), jnp.int32),
            pltpu.VMEM((QH, D), jnp.float32),
            pltpu.VMEM((QH, D), jnp.float32),
            pltpu.VMEM((QH, 2 * D), jnp.float32),
            pltpu.SemaphoreType.DMA,
        )


@jax.jit
def kernel(mem, dst_ids, msgs, query_ids):
    mesh = plsc.VectorSubcoreMesh(core_axis_name="c", subcore_axis_name="s")
    out, _ = pl.kernel(
        _body,
        out_type=(
            jax.ShapeDtypeStruct((B, 2 * D), jnp.float32),
            jax.ShapeDtypeStruct((NC * MP,), jnp.int32),
        ),
        mesh=mesh,
        scratch_types=[pltpu.VMEM_SHARED((B,), jnp.int32)],
        compiler_params=pltpu.CompilerParams(
            needs_layout_passes=False, use_tc_tiling_on_sc=True),
    )(mem, dst_ids, msgs, query_ids)
    return out[:, :D]
